# baseline (speedup 1.0000x reference)
import functools

import jax
import jax.numpy as jnp
from jax import lax
from jax.experimental import pallas as pl
from jax.experimental.pallas import tpu as pltpu
from jax.experimental.pallas import tpu_sc as plsc

_NC = 2
_NS = 16
_LANES = 16
_CHUNK = 1
_S_SC = 32
_BS = 224


def _tc_add_body(x_ref, pe_ref, o_ref):
    o_ref[...] = x_ref[...] + pe_ref[...][:, None, :]


def _make_sc_kernel(S_sc, base_row, B, D, dtype):
    NW = _NC * _NS
    rows_per_w = S_sc // NW
    n_chunks = rows_per_w // _CHUNK
    mesh = plsc.VectorSubcoreMesh(core_axis_name="c", subcore_axis_name="s")

    @functools.partial(
        pl.kernel,
        out_type=jax.ShapeDtypeStruct((S_sc, B, D), dtype),
        mesh=mesh,
        scratch_types=[
            pltpu.VMEM((_CHUNK, B, D), dtype),
            pltpu.VMEM((_CHUNK, B, D), dtype),
            pltpu.VMEM((_CHUNK, D), dtype),
            pltpu.VMEM((_CHUNK, D), dtype),
            pltpu.VMEM((_CHUNK, B, D), dtype),
            pltpu.VMEM((_CHUNK, B, D), dtype),
            pltpu.SemaphoreType.DMA,
            pltpu.SemaphoreType.DMA,
            pltpu.SemaphoreType.DMA,
            pltpu.SemaphoreType.DMA,
            pltpu.SemaphoreType.DMA,
            pltpu.SemaphoreType.DMA,
        ],
    )
    def k(x_hbm, pe_hbm, out_hbm, xb0, xb1, pb0, pb1, ob0, ob1,
          si0, si1, sp0, sp1, so0, so1):
        wid = lax.axis_index("s") * _NC + lax.axis_index("c")
        obase = wid * rows_per_w
        ibase = base_row + obase
        xbufs, pbufs, obufs = [xb0, xb1], [pb0, pb1], [ob0, ob1]
        sin, spe, sout = [si0, si1], [sp0, sp1], [so0, so1]
        in_d = [None] * n_chunks
        pe_d = [None] * n_chunks
        out_d = [None] * n_chunks

        def start_in(ci):
            row = ibase + ci * _CHUNK
            b = ci % 2
            in_d[ci] = pltpu.async_copy(
                x_hbm.at[pl.ds(row, _CHUNK)], xbufs[b], sin[b])
            pe_d[ci] = pltpu.async_copy(
                pe_hbm.at[pl.ds(row, _CHUNK)], pbufs[b], spe[b])

        start_in(0)
        for ci in range(n_chunks):
            b = ci % 2
            if ci + 1 < n_chunks:
                start_in(ci + 1)
            in_d[ci].wait()
            pe_d[ci].wait()
            if ci >= 2:
                out_d[ci - 2].wait()
            xbuf, pbuf, obuf = xbufs[b], pbufs[b], obufs[b]
            for r in range(_CHUNK):
                @plsc.parallel_loop(0, D, _LANES, unroll=4)
                def d_body(dd, r=r, xbuf=xbuf, pbuf=pbuf, obuf=obuf):
                    sl = pl.ds(dd, _LANES)
                    pv = pbuf[r, sl]
                    for bb in range(B):
                        obuf[r, bb, sl] = xbuf[r, bb, sl] + pv
            out_d[ci] = pltpu.async_copy(
                obuf, out_hbm.at[pl.ds(obase + ci * _CHUNK, _CHUNK)], sout[b])
        if n_chunks >= 2:
            out_d[n_chunks - 2].wait()
        out_d[n_chunks - 1].wait()

    return k


def kernel(x, pe):
    S, B, D = x.shape
    pe = pe[:S]
    s_tc = S - _S_SC

    sc_out = _make_sc_kernel(_S_SC, s_tc, B, D, x.dtype)(x, pe)

    tc_full = pl.pallas_call(
        _tc_add_body,
        grid=(s_tc // _BS,),
        in_specs=[
            pl.BlockSpec((_BS, B, D), lambda i: (i, 0, 0)),
            pl.BlockSpec((_BS, D), lambda i: (i, 0)),
        ],
        out_specs=pl.BlockSpec((_BS, B, D), lambda i: (i, 0, 0)),
        out_shape=jax.ShapeDtypeStruct((S, B, D), x.dtype),
    )(x, pe)

    return lax.dynamic_update_slice(tc_full, sc_out, (s_tc, 0, 0))


# speedup vs baseline: 1.0015x; 1.0015x over previous
import functools

import jax
import jax.numpy as jnp
from jax import lax
from jax.experimental import pallas as pl
from jax.experimental.pallas import tpu as pltpu
from jax.experimental.pallas import tpu_sc as plsc

_NC = 2
_NS = 16
_LANES = 16
_CHUNK = 1
_S_SC = 32
_BS = 224


def _tc_add_body(x_ref, pe_ref, o_ref):
    o_ref[...] = x_ref[...] + pe_ref[...][:, None, :]


def _merge_body(tc_ref, sc_ref, o_ref):
    del tc_ref
    o_ref[...] = sc_ref[...]


def _make_sc_kernel(S_sc, base_row, B, D, dtype):
    NW = _NC * _NS
    rows_per_w = S_sc // NW
    n_chunks = rows_per_w // _CHUNK
    mesh = plsc.VectorSubcoreMesh(core_axis_name="c", subcore_axis_name="s")

    @functools.partial(
        pl.kernel,
        out_type=jax.ShapeDtypeStruct((S_sc, B, D), dtype),
        mesh=mesh,
        scratch_types=[
            pltpu.VMEM((_CHUNK, B, D), dtype),
            pltpu.VMEM((_CHUNK, B, D), dtype),
            pltpu.VMEM((_CHUNK, D), dtype),
            pltpu.VMEM((_CHUNK, D), dtype),
            pltpu.VMEM((_CHUNK, B, D), dtype),
            pltpu.VMEM((_CHUNK, B, D), dtype),
            pltpu.SemaphoreType.DMA,
            pltpu.SemaphoreType.DMA,
            pltpu.SemaphoreType.DMA,
            pltpu.SemaphoreType.DMA,
            pltpu.SemaphoreType.DMA,
            pltpu.SemaphoreType.DMA,
        ],
    )
    def k(x_hbm, pe_hbm, out_hbm, xb0, xb1, pb0, pb1, ob0, ob1,
          si0, si1, sp0, sp1, so0, so1):
        wid = lax.axis_index("s") * _NC + lax.axis_index("c")
        obase = wid * rows_per_w
        ibase = base_row + obase
        xbufs, pbufs, obufs = [xb0, xb1], [pb0, pb1], [ob0, ob1]
        sin, spe, sout = [si0, si1], [sp0, sp1], [so0, so1]
        in_d = [None] * n_chunks
        pe_d = [None] * n_chunks
        out_d = [None] * n_chunks

        def start_in(ci):
            row = ibase + ci * _CHUNK
            b = ci % 2
            in_d[ci] = pltpu.async_copy(
                x_hbm.at[pl.ds(row, _CHUNK)], xbufs[b], sin[b])
            pe_d[ci] = pltpu.async_copy(
                pe_hbm.at[pl.ds(row, _CHUNK)], pbufs[b], spe[b])

        start_in(0)
        for ci in range(n_chunks):
            b = ci % 2
            if ci + 1 < n_chunks:
                start_in(ci + 1)
            in_d[ci].wait()
            pe_d[ci].wait()
            if ci >= 2:
                out_d[ci - 2].wait()
            xbuf, pbuf, obuf = xbufs[b], pbufs[b], obufs[b]
            for r in range(_CHUNK):
                @plsc.parallel_loop(0, D, _LANES, unroll=4)
                def d_body(dd, r=r, xbuf=xbuf, pbuf=pbuf, obuf=obuf):
                    sl = pl.ds(dd, _LANES)
                    pv = pbuf[r, sl]
                    for bb in range(B):
                        obuf[r, bb, sl] = xbuf[r, bb, sl] + pv
            out_d[ci] = pltpu.async_copy(
                obuf, out_hbm.at[pl.ds(obase + ci * _CHUNK, _CHUNK)], sout[b])
        if n_chunks >= 2:
            out_d[n_chunks - 2].wait()
        out_d[n_chunks - 1].wait()

    return k


def kernel(x, pe):
    S, B, D = x.shape
    pe = pe[:S]
    s_tc = S - _S_SC

    sc_out = _make_sc_kernel(_S_SC, s_tc, B, D, x.dtype)(x, pe)

    tc_full = pl.pallas_call(
        _tc_add_body,
        grid=(s_tc // _BS,),
        in_specs=[
            pl.BlockSpec((_BS, B, D), lambda i: (i, 0, 0)),
            pl.BlockSpec((_BS, D), lambda i: (i, 0)),
        ],
        out_specs=pl.BlockSpec((_BS, B, D), lambda i: (i, 0, 0)),
        out_shape=jax.ShapeDtypeStruct((S, B, D), x.dtype),
    )(x, pe)

    return pl.pallas_call(
        _merge_body,
        grid=(1,),
        in_specs=[
            pl.BlockSpec((_S_SC, B, D), lambda i: (s_tc // _S_SC, 0, 0)),
            pl.BlockSpec((_S_SC, B, D), lambda i: (0, 0, 0)),
        ],
        out_specs=pl.BlockSpec((_S_SC, B, D), lambda i: (s_tc // _S_SC, 0, 0)),
        out_shape=jax.ShapeDtypeStruct((S, B, D), x.dtype),
        input_output_aliases={0: 0},
    )(tc_full, sc_out)


# speedup vs baseline: 1.7009x; 1.6983x over previous
import jax
import jax.numpy as jnp
from jax.experimental import pallas as pl

_BS = 256


def _add_body(x_ref, pe_ref, o_ref):
    o_ref[...] = x_ref[...] + pe_ref[...][:, None, :]


def kernel(x, pe):
    S, B, D = x.shape
    pe = pe[:S]
    bs = _BS if S % _BS == 0 else pl.cdiv(S, pl.cdiv(S, _BS))
    if S % bs != 0:
        bs = S
    return pl.pallas_call(
        _add_body,
        grid=(S // bs,),
        in_specs=[
            pl.BlockSpec((bs, B, D), lambda i: (i, 0, 0)),
            pl.BlockSpec((bs, D), lambda i: (i, 0)),
        ],
        out_specs=pl.BlockSpec((bs, B, D), lambda i: (i, 0, 0)),
        out_shape=jax.ShapeDtypeStruct((S, B, D), x.dtype),
    )(x, pe)


# speedup vs baseline: 1.7010x; 1.0001x over previous
import jax
import jax.numpy as jnp
from jax.experimental import pallas as pl

_BS = 512


def _add_body(x_ref, pe_ref, o_ref):
    o_ref[...] = x_ref[...] + pe_ref[...][:, None, :]


def kernel(x, pe):
    S, B, D = x.shape
    pe = pe[:S]
    bs = _BS if S % _BS == 0 else pl.cdiv(S, pl.cdiv(S, _BS))
    if S % bs != 0:
        bs = S
    return pl.pallas_call(
        _add_body,
        grid=(S // bs,),
        in_specs=[
            pl.BlockSpec((bs, B, D), lambda i: (i, 0, 0)),
            pl.BlockSpec((bs, D), lambda i: (i, 0)),
        ],
        out_specs=pl.BlockSpec((bs, B, D), lambda i: (i, 0, 0)),
        out_shape=jax.ShapeDtypeStruct((S, B, D), x.dtype),
    )(x, pe)
